# Initial kernel scaffold; baseline (speedup 1.0000x reference)
import jax
import jax.numpy as jnp
from jax.experimental import pallas as pl


def kernel(x, edge_index, W, b):
    raise NotImplementedError("write your pallas kernel here")


# speedup vs baseline: 9.0313x; 9.0313x over previous
import functools

import jax
import jax.numpy as jnp
from jax import lax
from jax.experimental import pallas as pl
from jax.experimental.pallas import tpu as pltpu
from jax.experimental.pallas import tpu_sc as plsc

NC = 2
NS = 16
CW = 128
BR = 2
ZR = 56


def _ceil_to(a, m):
    return -(-a // m) * m


def _edge_grid(e_padded):
    rows = e_padded // CW
    ptr = rows // NS
    nblk = ptr // BR
    return ptr, nblk


def _compute_dloc(dbuf, dloc, base, nh, trash):
    for r in range(BR):
        for k in range(CW // 16):
            dv = dbuf[r, pl.ds(k * 16, 16)]
            loc = dv - base
            ok = (loc >= 0) & (loc < nh)
            dloc[r, pl.ds(k * 16, 16)] = jnp.where(ok, loc, trash)


def _make_deg_kernel(rows, n, nh, acc_rows, tpt):
    ptr, nblk = _edge_grid(rows * CW)
    mesh = plsc.VectorSubcoreMesh(core_axis_name="c", subcore_axis_name="s")

    @functools.partial(
        pl.kernel,
        out_type=jax.ShapeDtypeStruct((NC, acc_rows, 16), jnp.float32),
        mesh=mesh,
        scratch_types=[
            pltpu.VMEM_SHARED((acc_rows, 16), jnp.float32),
            pltpu.VMEM((BR, CW), jnp.int32),
            pltpu.VMEM((BR, CW), jnp.int32),
            pltpu.VMEM((CW, 16), jnp.float32),
            pltpu.VMEM((ZR, 16), jnp.float32),
        ],
        compiler_params=pltpu.CompilerParams(use_tc_tiling_on_sc=False),
    )
    def deg_kernel(dst_hbm, out_hbm, acc, dbuf, dloc, ones_v, zbuf):
        c = lax.axis_index("c")
        s = lax.axis_index("s")
        base = c * nh

        def _fill(i, _):
            zbuf[i, :] = jnp.zeros((16,), jnp.float32)
            return _

        lax.fori_loop(0, ZR, _fill, 0)

        def _fill1(i, _):
            ones_v[i, :] = jnp.ones((16,), jnp.float32)
            return _

        lax.fori_loop(0, CW, _fill1, 0)

        for j in range(tpt // ZR):
            pltpu.sync_copy(zbuf, acc.at[pl.ds(s * tpt + j * ZR, ZR)])
        plsc.subcore_barrier()

        def blk(bidx, _):
            row0 = s * ptr + bidx * BR
            pltpu.sync_copy(dst_hbm.at[pl.ds(row0, BR)], dbuf)
            _compute_dloc(dbuf, dloc, base, nh, nh)
            for r in range(BR):
                pltpu.sync_copy(ones_v, acc.at[dloc.at[r]], add=True)
            return _

        lax.fori_loop(0, nblk, blk, 0)
        plsc.subcore_barrier()
        pltpu.sync_copy(acc.at[pl.ds(s * tpt, tpt)],
                        out_hbm.at[c, pl.ds(s * tpt, tpt)])

    return deg_kernel


def _make_agg_kernel(rows, n, d, nh, acc_rows, tpt):
    ptr, nblk = _edge_grid(rows * CW)
    mesh = plsc.VectorSubcoreMesh(core_axis_name="c", subcore_axis_name="s")

    @functools.partial(
        pl.kernel,
        out_type=jax.ShapeDtypeStruct((NC, acc_rows, d), jnp.float32),
        mesh=mesh,
        scratch_types=[
            pltpu.VMEM_SHARED((acc_rows, d), jnp.float32),
            pltpu.VMEM((BR, CW), jnp.int32),
            pltpu.VMEM((BR, CW), jnp.int32),
            pltpu.VMEM((BR, CW), jnp.int32),
            pltpu.VMEM((BR, CW, d), jnp.float32),
            pltpu.VMEM((ZR, d), jnp.float32),
            pltpu.SemaphoreType.DMA,
        ],
        compiler_params=pltpu.CompilerParams(use_tc_tiling_on_sc=False),
    )
    def agg_kernel(src_hbm, dst_hbm, hs_hbm, out_hbm,
                   acc, sbuf, dbuf, dloc, rows_v, zbuf, sem):
        c = lax.axis_index("c")
        s = lax.axis_index("s")
        base = c * nh

        def _fill(i, _):
            for k in range(d // 16):
                zbuf[i, pl.ds(k * 16, 16)] = jnp.zeros((16,), jnp.float32)
            return _

        lax.fori_loop(0, ZR, _fill, 0)

        for j in range(tpt // ZR):
            pltpu.sync_copy(zbuf, acc.at[pl.ds(s * tpt + j * ZR, ZR)])
        plsc.subcore_barrier()

        def blk(bidx, _):
            row0 = s * ptr + bidx * BR
            pltpu.sync_copy(src_hbm.at[pl.ds(row0, BR)], sbuf)
            pltpu.sync_copy(dst_hbm.at[pl.ds(row0, BR)], dbuf)
            descs = [
                pltpu.async_copy(hs_hbm.at[sbuf.at[r]], rows_v.at[r], sem)
                for r in range(BR)
            ]
            _compute_dloc(dbuf, dloc, base, nh, nh)
            for r in range(BR):
                descs[r].wait()
            for r in range(BR):
                pltpu.sync_copy(rows_v.at[r], acc.at[dloc.at[r]], add=True)
            return _

        lax.fori_loop(0, nblk, blk, 0)
        plsc.subcore_barrier()
        pltpu.sync_copy(acc.at[pl.ds(s * tpt, tpt)],
                        out_hbm.at[c, pl.ds(s * tpt, tpt)])

    return agg_kernel


def _tc_first(deg, x, w, blk):
    n, d = x.shape

    def body(deg_ref, x_ref, w_ref, dinv_ref, hs_ref):
        dg = deg_ref[...]
        dinv = jnp.where(dg > 0, lax.rsqrt(dg), 0.0)
        dinv_ref[...] = dinv
        hs_ref[...] = jnp.dot(x_ref[...], w_ref[...],
                              preferred_element_type=jnp.float32) * dinv

    return pl.pallas_call(
        body,
        grid=(n // blk,),
        in_specs=[
            pl.BlockSpec((blk, 1), lambda i: (i, 0)),
            pl.BlockSpec((blk, d), lambda i: (i, 0)),
            pl.BlockSpec((d, d), lambda i: (0, 0)),
        ],
        out_specs=[
            pl.BlockSpec((blk, 1), lambda i: (i, 0)),
            pl.BlockSpec((blk, d), lambda i: (i, 0)),
        ],
        out_shape=[
            jax.ShapeDtypeStruct((n, 1), jnp.float32),
            jax.ShapeDtypeStruct((n, d), jnp.float32),
        ],
    )(deg, x, w)


def _tc_mid(agg_raw, dinv, bias, res, w, scale, blk):
    n, d = agg_raw.shape

    def body(agg_ref, dinv_ref, b_ref, res_ref, w_ref, out_ref, hs_ref):
        dv = dinv_ref[...]
        agg = agg_ref[...] * dv + b_ref[...]
        nrm = jnp.sqrt(jnp.sum(agg * agg, axis=-1, keepdims=True))
        h = agg / jnp.maximum(nrm, 1e-12)
        out_ref[...] = res_ref[...] + h * scale
        hs_ref[...] = jnp.dot(h, w_ref[...],
                              preferred_element_type=jnp.float32) * dv

    return pl.pallas_call(
        body,
        grid=(n // blk,),
        in_specs=[
            pl.BlockSpec((blk, d), lambda i: (i, 0)),
            pl.BlockSpec((blk, 1), lambda i: (i, 0)),
            pl.BlockSpec((1, d), lambda i: (0, 0)),
            pl.BlockSpec((blk, d), lambda i: (i, 0)),
            pl.BlockSpec((d, d), lambda i: (0, 0)),
        ],
        out_specs=[
            pl.BlockSpec((blk, d), lambda i: (i, 0)),
            pl.BlockSpec((blk, d), lambda i: (i, 0)),
        ],
        out_shape=[
            jax.ShapeDtypeStruct((n, d), jnp.float32),
            jax.ShapeDtypeStruct((n, d), jnp.float32),
        ],
    )(agg_raw, dinv, bias, res, w)


def _tc_last(agg_raw, dinv, bias, res, scale, blk):
    n, d = agg_raw.shape

    def body(agg_ref, dinv_ref, b_ref, res_ref, out_ref):
        agg = agg_ref[...] * dinv_ref[...] + b_ref[...]
        nrm = jnp.sqrt(jnp.sum(agg * agg, axis=-1, keepdims=True))
        h = agg / jnp.maximum(nrm, 1e-12)
        out_ref[...] = res_ref[...] + h * scale

    return pl.pallas_call(
        body,
        grid=(n // blk,),
        in_specs=[
            pl.BlockSpec((blk, d), lambda i: (i, 0)),
            pl.BlockSpec((blk, 1), lambda i: (i, 0)),
            pl.BlockSpec((1, d), lambda i: (0, 0)),
            pl.BlockSpec((blk, d), lambda i: (i, 0)),
        ],
        out_specs=pl.BlockSpec((blk, d), lambda i: (i, 0)),
        out_shape=jax.ShapeDtypeStruct((n, d), jnp.float32),
    )(agg_raw, dinv, bias, res)


def kernel(x, edge_index, W, b):
    n, d = x.shape
    num_layers = W.shape[0]
    e = edge_index.shape[1]
    nh = n // NC
    tpt = _ceil_to(nh + 8, NS) // NS
    tpt = _ceil_to(tpt, ZR)
    acc_rows = NS * tpt

    src = edge_index[0].astype(jnp.int32)
    dst = edge_index[1].astype(jnp.int32)
    ep = _ceil_to(e, CW * BR * NS)
    pad = ep - e
    rows = ep // CW
    src2d = jnp.concatenate([src, jnp.zeros((pad,), jnp.int32)]).reshape(rows, CW)
    dst2d = jnp.concatenate([dst, jnp.full((pad,), n, jnp.int32)]).reshape(rows, CW)

    deg16 = _make_deg_kernel(rows, n, nh, acc_rows, tpt)(dst2d)
    deg = deg16[:, :nh, 0].reshape(n, 1)

    blk = 2000
    agg_call = _make_agg_kernel(rows, n, d, nh, acc_rows, tpt)
    dinv, hs = _tc_first(deg, x, W[0], blk)
    result = x
    for i in range(num_layers):
        agg_raw = agg_call(src2d, dst2d, hs)[:, :nh, :].reshape(n, d)
        scale = 1.0 / (i + 1)
        if i + 1 < num_layers:
            result, hs = _tc_mid(agg_raw, dinv, b[i].reshape(1, d), result,
                                 W[i + 1], scale, blk)
        else:
            result = _tc_last(agg_raw, dinv, b[i].reshape(1, d), result,
                              scale, blk)
    return result


# speedup vs baseline: 9.0976x; 1.0074x over previous
import functools

import jax
import jax.numpy as jnp
from jax import lax
from jax.experimental import pallas as pl
from jax.experimental.pallas import tpu as pltpu
from jax.experimental.pallas import tpu_sc as plsc

NC = 2
NS = 16
CW = 128
NSLOT = 3
ZR = 28


def _ceil_to(a, m):
    return -(-a // m) * m


def _dloc_chunk(dbuf, dloc, slot, base, nh, trash):
    for k in range(CW // 16):
        dv = dbuf[slot, pl.ds(k * 16, 16)]
        loc = dv - base
        ok = (loc >= 0) & (loc < nh)
        dloc[slot, pl.ds(k * 16, 16)] = jnp.where(ok, loc, trash)


def _zero_acc(acc, zbuf, s, tpt, width):
    def _fill(i, carry):
        for k in range(width // 16):
            zbuf[i, pl.ds(k * 16, 16)] = jnp.zeros((16,), jnp.float32)
        return carry

    lax.fori_loop(0, ZR, _fill, 0)
    for j in range(tpt // ZR):
        pltpu.sync_copy(zbuf, acc.at[pl.ds(s * tpt + j * ZR, ZR)])


def _make_deg_kernel(rows, n, nh, acc_rows, tpt):
    ptr = rows // NS
    nblk = ptr
    mesh = plsc.VectorSubcoreMesh(core_axis_name="c", subcore_axis_name="s")

    @functools.partial(
        pl.kernel,
        out_type=jax.ShapeDtypeStruct((NC, acc_rows, 16), jnp.float32),
        mesh=mesh,
        scratch_types=[
            pltpu.VMEM_SHARED((acc_rows, 16), jnp.float32),
            pltpu.VMEM((NSLOT, CW), jnp.int32),
            pltpu.VMEM((NSLOT, CW), jnp.int32),
            pltpu.VMEM((CW, 16), jnp.float32),
            pltpu.VMEM((ZR, 16), jnp.float32),
            pltpu.SemaphoreType.DMA((NSLOT,)),
            pltpu.SemaphoreType.DMA((NSLOT,)),
        ],
        compiler_params=pltpu.CompilerParams(use_tc_tiling_on_sc=False),
    )
    def deg_kernel(dst_hbm, out_hbm, acc, dbuf, dloc, ones_v, zbuf,
                   idx_sem, sct_sem):
        c = lax.axis_index("c")
        s = lax.axis_index("s")
        base = c * nh
        row0 = s * ptr

        def _fill1(i, carry):
            ones_v[i, :] = jnp.ones((16,), jnp.float32)
            return carry

        lax.fori_loop(0, CW, _fill1, 0)
        _zero_acc(acc, zbuf, s, tpt, 16)
        plsc.subcore_barrier()

        def idx_start(b, slot):
            pltpu.async_copy(dst_hbm.at[row0 + b], dbuf.at[slot],
                             idx_sem.at[slot])

        def idx_wait(slot):
            pltpu.make_async_copy(dst_hbm.at[row0], dbuf.at[slot],
                                  idx_sem.at[slot]).wait()

        def sct_start(slot):
            pltpu.async_copy(ones_v, acc.at[dloc.at[slot]],
                             sct_sem.at[slot], add=True)

        def sct_wait(slot):
            pltpu.make_async_copy(ones_v, acc.at[dloc.at[slot]],
                                  sct_sem.at[slot]).wait()

        idx_start(0, 0)

        def body(g, carry):
            for k in range(NSLOT):
                b = g * NSLOT + k
                cur = k
                nxt = (k + 1) % NSLOT

                @pl.when(b + 1 < nblk)
                def _():
                    idx_start(b + 1, nxt)

                idx_wait(cur)

                @pl.when(b >= NSLOT)
                def _():
                    sct_wait(cur)

                _dloc_chunk(dbuf, dloc, cur, base, nh, nh)
                sct_start(cur)
            return carry

        lax.fori_loop(0, nblk // NSLOT, body, 0)
        for k in range(NSLOT):
            sct_wait(k)
        plsc.subcore_barrier()
        pltpu.sync_copy(acc.at[pl.ds(s * tpt, tpt)],
                        out_hbm.at[c, pl.ds(s * tpt, tpt)])

    return deg_kernel


def _make_agg_kernel(rows, n, d, nh, acc_rows, tpt):
    ptr = rows // NS
    nblk = ptr
    mesh = plsc.VectorSubcoreMesh(core_axis_name="c", subcore_axis_name="s")

    @functools.partial(
        pl.kernel,
        out_type=jax.ShapeDtypeStruct((NC, acc_rows, d), jnp.float32),
        mesh=mesh,
        scratch_types=[
            pltpu.VMEM_SHARED((acc_rows, d), jnp.float32),
            pltpu.VMEM((NSLOT, CW), jnp.int32),
            pltpu.VMEM((NSLOT, CW), jnp.int32),
            pltpu.VMEM((NSLOT, CW), jnp.int32),
            pltpu.VMEM((NSLOT, CW, d), jnp.float32),
            pltpu.VMEM((ZR, d), jnp.float32),
            pltpu.SemaphoreType.DMA((NSLOT,)),
            pltpu.SemaphoreType.DMA((NSLOT,)),
            pltpu.SemaphoreType.DMA((NSLOT,)),
        ],
        compiler_params=pltpu.CompilerParams(use_tc_tiling_on_sc=False),
    )
    def agg_kernel(src_hbm, dst_hbm, hs_hbm, out_hbm,
                   acc, sbuf, dbuf, dloc, rows_v, zbuf,
                   idx_sem, gat_sem, sct_sem):
        c = lax.axis_index("c")
        s = lax.axis_index("s")
        base = c * nh
        row0 = s * ptr

        _zero_acc(acc, zbuf, s, tpt, d)
        plsc.subcore_barrier()

        def idx_start(b, slot):
            pltpu.async_copy(src_hbm.at[row0 + b], sbuf.at[slot],
                             idx_sem.at[slot])
            pltpu.async_copy(dst_hbm.at[row0 + b], dbuf.at[slot],
                             idx_sem.at[slot])

        def idx_wait(slot):
            pltpu.make_async_copy(src_hbm.at[row0], sbuf.at[slot],
                                  idx_sem.at[slot]).wait()
            pltpu.make_async_copy(dst_hbm.at[row0], dbuf.at[slot],
                                  idx_sem.at[slot]).wait()

        def gat_start(slot):
            pltpu.async_copy(hs_hbm.at[sbuf.at[slot]], rows_v.at[slot],
                             gat_sem.at[slot])

        def gat_wait(slot):
            pltpu.make_async_copy(hs_hbm.at[sbuf.at[slot]], rows_v.at[slot],
                                  gat_sem.at[slot]).wait()

        def sct_start(slot):
            pltpu.async_copy(rows_v.at[slot], acc.at[dloc.at[slot]],
                             sct_sem.at[slot], add=True)

        def sct_wait(slot):
            pltpu.make_async_copy(rows_v.at[slot], acc.at[dloc.at[slot]],
                                  sct_sem.at[slot]).wait()

        idx_start(0, 0)
        idx_start(1, 1)
        idx_wait(0)
        _dloc_chunk(dbuf, dloc, 0, base, nh, nh)
        gat_start(0)

        def body(g, carry):
            for k in range(NSLOT):
                b = g * NSLOT + k
                cur = k
                nxt = (k + 1) % NSLOT
                pf = (k + 2) % NSLOT

                @pl.when(b + 2 < nblk)
                def _():
                    idx_start(b + 2, pf)

                @pl.when(b + 1 < nblk)
                def _():
                    idx_wait(nxt)

                    @pl.when(b >= 2)
                    def _():
                        sct_wait(nxt)

                    _dloc_chunk(dbuf, dloc, nxt, base, nh, nh)
                    gat_start(nxt)

                gat_wait(cur)
                sct_start(cur)
            return carry

        lax.fori_loop(0, nblk // NSLOT, body, 0)
        for k in range(NSLOT):
            sct_wait(k)
        plsc.subcore_barrier()
        pltpu.sync_copy(acc.at[pl.ds(s * tpt, tpt)],
                        out_hbm.at[c, pl.ds(s * tpt, tpt)])

    return agg_kernel


def _tc_first(deg16, x, w, blk, nh):
    n, d = x.shape
    nb = nh // blk

    def body(deg_ref, x_ref, w_ref, dinv_ref, hs_ref):
        dg = deg_ref[...][0, :, 0:1]
        dinv = jnp.where(dg > 0, lax.rsqrt(dg), 0.0)
        dinv_ref[...] = dinv
        hs_ref[...] = jnp.dot(x_ref[...], w_ref[...],
                              preferred_element_type=jnp.float32) * dinv

    return pl.pallas_call(
        body,
        grid=(n // blk,),
        in_specs=[
            pl.BlockSpec((1, blk, 16), lambda i: (i // nb, i % nb, 0)),
            pl.BlockSpec((blk, d), lambda i: (i, 0)),
            pl.BlockSpec((d, d), lambda i: (0, 0)),
        ],
        out_specs=[
            pl.BlockSpec((blk, 1), lambda i: (i, 0)),
            pl.BlockSpec((blk, d), lambda i: (i, 0)),
        ],
        out_shape=[
            jax.ShapeDtypeStruct((n, 1), jnp.float32),
            jax.ShapeDtypeStruct((n, d), jnp.float32),
        ],
    )(deg16, x, w)


def _tc_mid(agg3, dinv, bias, res, w, scale, blk, nh):
    n, d = res.shape
    nb = nh // blk

    def body(agg_ref, dinv_ref, b_ref, res_ref, w_ref, out_ref, hs_ref):
        dv = dinv_ref[...]
        agg = agg_ref[...][0] * dv + b_ref[...]
        nrm = jnp.sqrt(jnp.sum(agg * agg, axis=-1, keepdims=True))
        h = agg / jnp.maximum(nrm, 1e-12)
        out_ref[...] = res_ref[...] + h * scale
        hs_ref[...] = jnp.dot(h, w_ref[...],
                              preferred_element_type=jnp.float32) * dv

    return pl.pallas_call(
        body,
        grid=(n // blk,),
        in_specs=[
            pl.BlockSpec((1, blk, d), lambda i: (i // nb, i % nb, 0)),
            pl.BlockSpec((blk, 1), lambda i: (i, 0)),
            pl.BlockSpec((1, d), lambda i: (0, 0)),
            pl.BlockSpec((blk, d), lambda i: (i, 0)),
            pl.BlockSpec((d, d), lambda i: (0, 0)),
        ],
        out_specs=[
            pl.BlockSpec((blk, d), lambda i: (i, 0)),
            pl.BlockSpec((blk, d), lambda i: (i, 0)),
        ],
        out_shape=[
            jax.ShapeDtypeStruct((n, d), jnp.float32),
            jax.ShapeDtypeStruct((n, d), jnp.float32),
        ],
    )(agg3, dinv, bias, res, w)


def _tc_last(agg3, dinv, bias, res, scale, blk, nh):
    n, d = res.shape
    nb = nh // blk

    def body(agg_ref, dinv_ref, b_ref, res_ref, out_ref):
        agg = agg_ref[...][0] * dinv_ref[...] + b_ref[...]
        nrm = jnp.sqrt(jnp.sum(agg * agg, axis=-1, keepdims=True))
        h = agg / jnp.maximum(nrm, 1e-12)
        out_ref[...] = res_ref[...] + h * scale

    return pl.pallas_call(
        body,
        grid=(n // blk,),
        in_specs=[
            pl.BlockSpec((1, blk, d), lambda i: (i // nb, i % nb, 0)),
            pl.BlockSpec((blk, 1), lambda i: (i, 0)),
            pl.BlockSpec((1, d), lambda i: (0, 0)),
            pl.BlockSpec((blk, d), lambda i: (i, 0)),
        ],
        out_specs=pl.BlockSpec((blk, d), lambda i: (i, 0)),
        out_shape=jax.ShapeDtypeStruct((n, d), jnp.float32),
    )(agg3, dinv, bias, res)


def kernel(x, edge_index, W, b):
    n, d = x.shape
    num_layers = W.shape[0]
    e = edge_index.shape[1]
    nh = n // NC
    tpt = _ceil_to(nh + 8, NS) // NS
    tpt = _ceil_to(tpt, ZR)
    acc_rows = NS * tpt

    src = edge_index[0].astype(jnp.int32)
    dst = edge_index[1].astype(jnp.int32)
    rows = _ceil_to(-(-e // CW), NS * NSLOT)
    pad = rows * CW - e
    src2d = jnp.concatenate([src, jnp.zeros((pad,), jnp.int32)]).reshape(rows, CW)
    dst2d = jnp.concatenate([dst, jnp.full((pad,), n, jnp.int32)]).reshape(rows, CW)

    deg16 = _make_deg_kernel(rows, n, nh, acc_rows, tpt)(dst2d)

    blk = 1000
    agg_call = _make_agg_kernel(rows, n, d, nh, acc_rows, tpt)
    dinv, hs = _tc_first(deg16, x, W[0], blk, nh)
    result = x
    for i in range(num_layers):
        agg3 = agg_call(src2d, dst2d, hs)
        scale = 1.0 / (i + 1)
        if i + 1 < num_layers:
            result, hs = _tc_mid(agg3, dinv, b[i].reshape(1, d), result,
                                 W[i + 1], scale, blk, nh)
        else:
            result = _tc_last(agg3, dinv, b[i].reshape(1, d), result,
                              scale, blk, nh)
    return result


# speedup vs baseline: 19.3629x; 2.1283x over previous
import functools

import jax
import jax.numpy as jnp
from jax import lax
from jax.experimental import pallas as pl
from jax.experimental.pallas import tpu as pltpu
from jax.experimental.pallas import tpu_sc as plsc

NC = 2
NS = 16
CW = 128
NSLOT = 3
ZR = 28
STG = 144


def _ceil_to(a, m):
    return -(-a // m) * m


def _zero_acc(acc, zbuf, s, tpt, width):
    def _fill(i, carry):
        for k in range(width // 16):
            zbuf[i, pl.ds(k * 16, 16)] = jnp.zeros((16,), jnp.float32)
        return carry

    lax.fori_loop(0, ZR, _fill, 0)
    for j in range(tpt // ZR):
        pltpu.sync_copy(zbuf, acc.at[pl.ds(s * tpt + j * ZR, ZR)])


def _make_partition_kernel(rows, n, nh, tpt, capc):
    ptr = rows // NS
    mesh = plsc.VectorSubcoreMesh(core_axis_name="c", subcore_axis_name="s", num_cores=NC, num_subcores=NS)
    i32 = jnp.int32

    @functools.partial(
        pl.kernel,
        out_type=[
            jax.ShapeDtypeStruct((NC, NS, tpt, 16), jnp.float32),
            jax.ShapeDtypeStruct((NC, NS, capc, CW), i32),
            jax.ShapeDtypeStruct((NC, NS, capc, CW), i32),
            jax.ShapeDtypeStruct((NC, NS, 16), i32),
        ],
        mesh=mesh,
        scratch_types=[
            pltpu.VMEM((tpt, 16), jnp.float32),
            pltpu.VMEM((NSLOT, CW), i32),
            pltpu.VMEM((NSLOT, CW), i32),
            pltpu.VMEM((2 * STG,), i32),
            pltpu.VMEM((2 * STG,), i32),
            pltpu.VMEM((16,), i32),
            pltpu.SemaphoreType.DMA((NSLOT,)),
            pltpu.SemaphoreType.DMA((2,)),
        ],
        compiler_params=pltpu.CompilerParams(use_tc_tiling_on_sc=False,
                                             needs_layout_passes=False),
    )
    def part_kernel(src_hbm, dst_hbm, hist_out, bsrc, bdloc, counts,
                    hist, sbuf, dbuf, csrc, cdloc, cbuf, idx_sem, fsem):
        c = lax.axis_index("c")
        s = lax.axis_index("s")
        base = c * nh
        trash = nh
        row0 = s * ptr

        def _hz(i, carry):
            hist[i, :] = jnp.zeros((16,), jnp.float32)
            return carry

        lax.fori_loop(0, tpt, _hz, 0)

        def idx_start(b, slot):
            pltpu.async_copy(src_hbm.at[row0 + b], sbuf.at[slot],
                             idx_sem.at[slot])
            pltpu.async_copy(dst_hbm.at[row0 + b], dbuf.at[slot],
                             idx_sem.at[slot])

        def idx_wait(slot):
            pltpu.make_async_copy(src_hbm.at[row0], sbuf.at[slot],
                                  idx_sem.at[slot]).wait()
            pltpu.make_async_copy(dst_hbm.at[row0], dbuf.at[slot],
                                  idx_sem.at[slot]).wait()

        def flush_wait(ph):
            pltpu.make_async_copy(csrc.at[pl.ds(0, CW)], bsrc.at[c, s, 0],
                                  fsem.at[ph]).wait()
            pltpu.make_async_copy(csrc.at[pl.ds(0, CW)], bsrc.at[c, s, 0],
                                  fsem.at[ph]).wait()

        def flush_start(nchunk, ph, off):
            pltpu.async_copy(csrc.at[pl.ds(off, CW)],
                             bsrc.at[c, s, nchunk], fsem.at[ph])
            pltpu.async_copy(cdloc.at[pl.ds(off, CW)],
                             bdloc.at[c, s, nchunk], fsem.at[ph])

        idx_start(0, 0)
        idx_start(1, 1)

        def row_body(b, carry):
            pos, nchunk = carry

            @pl.when(b + 2 < ptr)
            def _():
                idx_start(b + 2, (b + 2) % NSLOT)

            slot = b % NSLOT
            idx_wait(slot)
            for k in range(CW // 16):
                sv = sbuf[slot, pl.ds(k * 16, 16)]
                dv = dbuf[slot, pl.ds(k * 16, 16)]
                loc = dv - base
                ok = (loc >= 0) & (loc < nh)
                oki = jnp.where(ok, 1, 0)
                incl = plsc.cumsum(oki)
                cnt = jnp.sum(oki)
                ph = nchunk & 1
                widx = ph * STG + pos + incl - 1
                plsc.store_scatter(csrc, [widx], sv, mask=ok)
                plsc.store_scatter(cdloc, [widx], loc, mask=ok)
                cnts, lastm = plsc.scan_count(loc, mask=ok)
                plsc.addupdate_scatter(
                    hist, [lax.shift_right_logical(loc, 4), loc & 15],
                    cnts.astype(jnp.float32), mask=lastm)
                pos = pos + cnt
                flushed = pos >= CW

                @pl.when(flushed)
                def _():
                    flush_start(nchunk, ph, ph * STG)

                    @pl.when(nchunk >= 1)
                    def _():
                        flush_wait(1 - ph)
                    tail = csrc[pl.ds(ph * STG + CW, 16)]
                    csrc[pl.ds((1 - ph) * STG, 16)] = tail
                    tail2 = cdloc[pl.ds(ph * STG + CW, 16)]
                    cdloc[pl.ds((1 - ph) * STG, 16)] = tail2

                pos = jnp.where(flushed, pos - CW, pos)
                nchunk = nchunk + jnp.where(flushed, 1, 0)
            return pos, nchunk

        pos, nchunk = lax.fori_loop(
            0, ptr, row_body, (jnp.int32(0), jnp.int32(0)))

        ph = nchunk & 1
        iota = lax.iota(i32, 16)
        for j in range(CW // 16):
            off = pos + j * 16
            tmask = (off + iota) < CW
            plsc.store_scatter(csrc, [ph * STG + off + iota],
                               jnp.zeros((16,), i32), mask=tmask)
            plsc.store_scatter(cdloc, [ph * STG + off + iota],
                               jnp.full((16,), trash, i32), mask=tmask)

        @pl.when(pos > 0)
        def _():
            flush_start(nchunk, ph, ph * STG)

            @pl.when(nchunk >= 1)
            def _():
                flush_wait(1 - ph)

        nchunk = nchunk + jnp.where(pos > 0, 1, 0)

        total = jnp.maximum(_sc_ceil3(nchunk), NSLOT)
        ph2 = nchunk & 1

        @pl.when(nchunk >= 1)
        def _():
            flush_wait(1 - ph2)

        for j in range(CW // 16):
            csrc[pl.ds(j * 16, 16)] = jnp.zeros((16,), i32)
            cdloc[pl.ds(j * 16, 16)] = jnp.full((16,), trash, i32)

        def pad_body(i, carry):
            pltpu.async_copy(csrc.at[pl.ds(0, CW)], bsrc.at[c, s, i],
                             fsem.at[i & 1])
            pltpu.async_copy(cdloc.at[pl.ds(0, CW)], bdloc.at[c, s, i],
                             fsem.at[i & 1])
            flush_wait(i & 1)
            return carry

        lax.fori_loop(nchunk, total, pad_body, 0)

        cbuf[...] = jnp.broadcast_to(total, (16,)).astype(i32)
        pltpu.sync_copy(cbuf, counts.at[c, s])
        pltpu.sync_copy(hist, hist_out.at[c, s])

    return part_kernel


def _sc_ceil3(x):
    return (x + 2) - lax.rem(x + 2, 3)


def _make_agg_kernel(n, d, nh, acc_rows, tpt, capc):
    mesh = plsc.VectorSubcoreMesh(core_axis_name="c", subcore_axis_name="s", num_cores=NC, num_subcores=NS)
    i32 = jnp.int32

    @functools.partial(
        pl.kernel,
        out_type=jax.ShapeDtypeStruct((NC, acc_rows, d), jnp.float32),
        mesh=mesh,
        scratch_types=[
            pltpu.VMEM_SHARED((acc_rows, d), jnp.float32),
            pltpu.VMEM((NSLOT, CW), i32),
            pltpu.VMEM((NSLOT, CW), i32),
            pltpu.VMEM((NSLOT, CW, d), jnp.float32),
            pltpu.VMEM((ZR, d), jnp.float32),
            pltpu.VMEM((16,), i32),
            pltpu.SemaphoreType.DMA((NSLOT,)),
            pltpu.SemaphoreType.DMA((NSLOT,)),
            pltpu.SemaphoreType.DMA((NSLOT,)),
        ],
        compiler_params=pltpu.CompilerParams(use_tc_tiling_on_sc=False),
    )
    def agg_kernel(bsrc, bdloc, cnts_hbm, hs_hbm, out_hbm,
                   acc, sbuf, dbuf, rows_v, zbuf, cbuf,
                   idx_sem, gat_sem, sct_sem):
        c = lax.axis_index("c")
        s = lax.axis_index("s")

        pltpu.sync_copy(cnts_hbm.at[c, s], cbuf)
        nblk = cbuf[...][0]

        _zero_acc(acc, zbuf, s, tpt, d)
        plsc.subcore_barrier()

        def idx_start(b, slot):
            pltpu.async_copy(bsrc.at[c, s, b], sbuf.at[slot],
                             idx_sem.at[slot])
            pltpu.async_copy(bdloc.at[c, s, b], dbuf.at[slot],
                             idx_sem.at[slot])

        def idx_wait(slot):
            pltpu.make_async_copy(bsrc.at[c, s, 0], sbuf.at[slot],
                                  idx_sem.at[slot]).wait()
            pltpu.make_async_copy(bdloc.at[c, s, 0], dbuf.at[slot],
                                  idx_sem.at[slot]).wait()

        def gat_start(slot):
            pltpu.async_copy(hs_hbm.at[sbuf.at[slot]], rows_v.at[slot],
                             gat_sem.at[slot])

        def gat_wait(slot):
            pltpu.make_async_copy(hs_hbm.at[sbuf.at[slot]], rows_v.at[slot],
                                  gat_sem.at[slot]).wait()

        def sct_start(slot):
            pltpu.async_copy(rows_v.at[slot], acc.at[dbuf.at[slot]],
                             sct_sem.at[slot], add=True)

        def sct_wait(slot):
            pltpu.make_async_copy(rows_v.at[slot], acc.at[dbuf.at[slot]],
                                  sct_sem.at[slot]).wait()

        idx_start(0, 0)
        idx_start(1, 1)
        idx_wait(0)
        gat_start(0)

        def body(g, carry):
            for k in range(NSLOT):
                b = g * NSLOT + k
                cur = k
                nxt = (k + 1) % NSLOT
                pf = (k + 2) % NSLOT

                @pl.when(b + 2 < nblk)
                def _():
                    idx_start(b + 2, pf)

                @pl.when(b + 1 < nblk)
                def _():
                    idx_wait(nxt)

                    @pl.when(b >= 2)
                    def _():
                        sct_wait(nxt)

                    gat_start(nxt)

                gat_wait(cur)
                sct_start(cur)
            return carry

        lax.fori_loop(0, nblk // NSLOT, body, 0)
        for k in range(NSLOT):
            sct_wait(k)
        plsc.subcore_barrier()
        pltpu.sync_copy(acc.at[pl.ds(s * tpt, tpt)],
                        out_hbm.at[c, pl.ds(s * tpt, tpt)])

    return agg_kernel


def _tc_dinv(hist, tpt, blkr):
    def body(h_ref, out_ref):
        dg = jnp.sum(h_ref[...], axis=1)
        out_ref[...] = jnp.where(dg > 0, lax.rsqrt(dg), 0.0)

    return pl.pallas_call(
        body,
        grid=(NC, tpt // blkr),
        in_specs=[pl.BlockSpec((1, NS, blkr, 16), lambda i, j: (i, 0, j, 0))],
        out_specs=pl.BlockSpec((1, blkr, 16), lambda i, j: (i, j, 0)),
        out_shape=jax.ShapeDtypeStruct((NC, tpt, 16), jnp.float32),
    )(hist)


def _tc_first(dinv, x, w, blk):
    n, d = x.shape

    def body(dinv_ref, x_ref, w_ref, hs_ref):
        hs_ref[...] = jnp.dot(x_ref[...], w_ref[...],
                              preferred_element_type=jnp.float32) * dinv_ref[...]

    return pl.pallas_call(
        body,
        grid=(n // blk,),
        in_specs=[
            pl.BlockSpec((blk, 1), lambda i: (i, 0)),
            pl.BlockSpec((blk, d), lambda i: (i, 0)),
            pl.BlockSpec((d, d), lambda i: (0, 0)),
        ],
        out_specs=pl.BlockSpec((blk, d), lambda i: (i, 0)),
        out_shape=jax.ShapeDtypeStruct((n, d), jnp.float32),
    )(dinv, x, w)


def _tc_mid(agg3, dinv, bias, res, w, scale, blk, nh):
    n, d = res.shape
    nb = nh // blk

    def body(agg_ref, dinv_ref, b_ref, res_ref, w_ref, out_ref, hs_ref):
        dv = dinv_ref[...]
        agg = agg_ref[...][0] * dv + b_ref[...]
        nrm = jnp.sqrt(jnp.sum(agg * agg, axis=-1, keepdims=True))
        h = agg / jnp.maximum(nrm, 1e-12)
        out_ref[...] = res_ref[...] + h * scale
        hs_ref[...] = jnp.dot(h, w_ref[...],
                              preferred_element_type=jnp.float32) * dv

    return pl.pallas_call(
        body,
        grid=(n // blk,),
        in_specs=[
            pl.BlockSpec((1, blk, d), lambda i: (i // nb, i % nb, 0)),
            pl.BlockSpec((blk, 1), lambda i: (i, 0)),
            pl.BlockSpec((1, d), lambda i: (0, 0)),
            pl.BlockSpec((blk, d), lambda i: (i, 0)),
            pl.BlockSpec((d, d), lambda i: (0, 0)),
        ],
        out_specs=[
            pl.BlockSpec((blk, d), lambda i: (i, 0)),
            pl.BlockSpec((blk, d), lambda i: (i, 0)),
        ],
        out_shape=[
            jax.ShapeDtypeStruct((n, d), jnp.float32),
            jax.ShapeDtypeStruct((n, d), jnp.float32),
        ],
    )(agg3, dinv, bias, res, w)


def _tc_last(agg3, dinv, bias, res, scale, blk, nh):
    n, d = res.shape
    nb = nh // blk

    def body(agg_ref, dinv_ref, b_ref, res_ref, out_ref):
        agg = agg_ref[...][0] * dinv_ref[...] + b_ref[...]
        nrm = jnp.sqrt(jnp.sum(agg * agg, axis=-1, keepdims=True))
        h = agg / jnp.maximum(nrm, 1e-12)
        out_ref[...] = res_ref[...] + h * scale

    return pl.pallas_call(
        body,
        grid=(n // blk,),
        in_specs=[
            pl.BlockSpec((1, blk, d), lambda i: (i // nb, i % nb, 0)),
            pl.BlockSpec((blk, 1), lambda i: (i, 0)),
            pl.BlockSpec((1, d), lambda i: (0, 0)),
            pl.BlockSpec((blk, d), lambda i: (i, 0)),
        ],
        out_specs=pl.BlockSpec((blk, d), lambda i: (i, 0)),
        out_shape=jax.ShapeDtypeStruct((n, d), jnp.float32),
    )(agg3, dinv, bias, res)


def kernel(x, edge_index, W, b):
    n, d = x.shape
    num_layers = W.shape[0]
    e = edge_index.shape[1]
    nh = n // NC
    tpt = _ceil_to(nh + 8, NS) // NS
    tpt = _ceil_to(tpt, ZR)
    acc_rows = NS * tpt

    src = edge_index[0].astype(jnp.int32)
    dst = edge_index[1].astype(jnp.int32)
    rows = _ceil_to(-(-e // CW), NS)
    pad = rows * CW - e
    src2d = jnp.concatenate([src, jnp.zeros((pad,), jnp.int32)]).reshape(rows, CW)
    dst2d = jnp.concatenate([dst, jnp.full((pad,), n, jnp.int32)]).reshape(rows, CW)

    ptr = rows // NS
    capc = _ceil_to(ptr + 1, NSLOT)

    hist, bsrc, bdloc, counts = _make_partition_kernel(
        rows, n, nh, tpt, capc)(src2d, dst2d)
    dinv3 = _tc_dinv(hist, tpt, tpt // 7)
    dinv = dinv3.reshape(NC, acc_rows)[:, :nh].reshape(n, 1)

    blk = 1000
    agg_call = _make_agg_kernel(n, d, nh, acc_rows, tpt, capc)
    hs = _tc_first(dinv, x, W[0], blk)
    result = x
    for i in range(num_layers):
        agg3 = agg_call(bsrc, bdloc, counts, hs)
        scale = 1.0 / (i + 1)
        if i + 1 < num_layers:
            result, hs = _tc_mid(agg3, dinv, b[i].reshape(1, d), result,
                                 W[i + 1], scale, blk, nh)
        else:
            result = _tc_last(agg3, dinv, b[i].reshape(1, d), result,
                              scale, blk, nh)
    return result


# speedup vs baseline: 19.7486x; 1.0199x over previous
import functools

import jax
import jax.numpy as jnp
from jax import lax
from jax.experimental import pallas as pl
from jax.experimental.pallas import tpu as pltpu
from jax.experimental.pallas import tpu_sc as plsc

NC = 2
NS = 16
CW = 128
NSLOT = 3
ZR = 28
STG = 144


def _ceil_to(a, m):
    return -(-a // m) * m


def _zero_acc(acc, zbuf, s, tpt, width):
    def _fill(i, carry):
        for k in range(width // 16):
            zbuf[i, pl.ds(k * 16, 16)] = jnp.zeros((16,), jnp.float32)
        return carry

    lax.fori_loop(0, ZR, _fill, 0)
    for j in range(tpt // ZR):
        pltpu.sync_copy(zbuf, acc.at[pl.ds(s * tpt + j * ZR, ZR)])


def _make_partition_kernel(rows, n, nh, tpt, capc):
    ptr = -(-rows // NS)
    tpr = tpt // 16
    mesh = plsc.VectorSubcoreMesh(core_axis_name="c", subcore_axis_name="s", num_cores=NC, num_subcores=NS)
    i32 = jnp.int32

    @functools.partial(
        pl.kernel,
        out_type=[
            jax.ShapeDtypeStruct((n,), jnp.float32),
            jax.ShapeDtypeStruct((NC, NS, capc, CW), i32),
            jax.ShapeDtypeStruct((NC, NS, capc, CW), i32),
            jax.ShapeDtypeStruct((NC, NS, 16), i32),
        ],
        mesh=mesh,
        scratch_types=[
            pltpu.VMEM_SHARED((NS, tpt, 16), jnp.float32),
            pltpu.VMEM((tpt, 16), jnp.float32),
            pltpu.VMEM((NS, tpr, 16), jnp.float32),
            pltpu.VMEM((tpt,), jnp.float32),
            pltpu.VMEM((NSLOT, CW), i32),
            pltpu.VMEM((NSLOT, CW), i32),
            pltpu.VMEM((2 * STG,), i32),
            pltpu.VMEM((2 * STG,), i32),
            pltpu.VMEM((16,), i32),
            pltpu.SemaphoreType.DMA((NSLOT,)),
            pltpu.SemaphoreType.DMA((2,)),
        ],
        compiler_params=pltpu.CompilerParams(use_tc_tiling_on_sc=False,
                                             needs_layout_passes=False),
    )
    def part_kernel(src_hbm, dst_hbm, dinv_out, bsrc, bdloc, counts,
                    hshared, hist, rbuf, dbuf2, sbuf, dbuf, csrc, cdloc,
                    cbuf, idx_sem, fsem):
        c = lax.axis_index("c")
        s = lax.axis_index("s")
        base = c * nh
        trash = nh
        nb_t = (rows - s + NS - 1) // NS

        def _hz(i, carry):
            hist[i, :] = jnp.zeros((16,), jnp.float32)
            return carry

        lax.fori_loop(0, tpt, _hz, 0)

        def idx_start(b, slot):
            pltpu.async_copy(src_hbm.at[b * NS + s], sbuf.at[slot],
                             idx_sem.at[slot])
            pltpu.async_copy(dst_hbm.at[b * NS + s], dbuf.at[slot],
                             idx_sem.at[slot])

        def idx_wait(slot):
            pltpu.make_async_copy(src_hbm.at[s], sbuf.at[slot],
                                  idx_sem.at[slot]).wait()
            pltpu.make_async_copy(dst_hbm.at[s], dbuf.at[slot],
                                  idx_sem.at[slot]).wait()

        def flush_wait(ph):
            pltpu.make_async_copy(csrc.at[pl.ds(0, CW)], bsrc.at[c, s, 0],
                                  fsem.at[ph]).wait()
            pltpu.make_async_copy(csrc.at[pl.ds(0, CW)], bsrc.at[c, s, 0],
                                  fsem.at[ph]).wait()

        def flush_start(nchunk, ph, off):
            pltpu.async_copy(csrc.at[pl.ds(off, CW)],
                             bsrc.at[c, s, nchunk], fsem.at[ph])
            pltpu.async_copy(cdloc.at[pl.ds(off, CW)],
                             bdloc.at[c, s, nchunk], fsem.at[ph])

        idx_start(0, 0)
        idx_start(1, 1)

        def row_body(b, carry):
            pos, nchunk = carry

            @pl.when(b + 2 < nb_t)
            def _():
                idx_start(b + 2, (b + 2) % NSLOT)

            slot = b % NSLOT
            idx_wait(slot)
            for k in range(CW // 16):
                sv = sbuf[slot, pl.ds(k * 16, 16)]
                dv = dbuf[slot, pl.ds(k * 16, 16)]
                loc = dv - base
                ok = (loc >= 0) & (loc < nh)
                oki = jnp.where(ok, 1, 0)
                incl = plsc.cumsum(oki)
                cnt = jnp.sum(oki)
                ph = nchunk & 1
                widx = ph * STG + pos + incl - 1
                plsc.store_scatter(csrc, [widx], sv, mask=ok)
                plsc.store_scatter(cdloc, [widx], loc, mask=ok)
                cnts, lastm = plsc.scan_count(loc, mask=ok)
                plsc.addupdate_scatter(
                    hist, [lax.shift_right_logical(loc, 4), loc & 15],
                    cnts.astype(jnp.float32), mask=lastm)
                pos = pos + cnt
                flushed = pos >= CW

                @pl.when(flushed)
                def _():
                    flush_start(nchunk, ph, ph * STG)

                    @pl.when(nchunk >= 1)
                    def _():
                        flush_wait(1 - ph)
                    tail = csrc[pl.ds(ph * STG + CW, 16)]
                    csrc[pl.ds((1 - ph) * STG, 16)] = tail
                    tail2 = cdloc[pl.ds(ph * STG + CW, 16)]
                    cdloc[pl.ds((1 - ph) * STG, 16)] = tail2

                pos = jnp.where(flushed, pos - CW, pos)
                nchunk = nchunk + jnp.where(flushed, 1, 0)
            return pos, nchunk

        pos, nchunk = lax.fori_loop(
            0, nb_t, row_body, (jnp.int32(0), jnp.int32(0)))

        ph = nchunk & 1
        iota = lax.iota(i32, 16)
        for j in range(CW // 16):
            off = pos + j * 16
            tmask = (off + iota) < CW
            plsc.store_scatter(csrc, [ph * STG + off + iota],
                               jnp.zeros((16,), i32), mask=tmask)
            plsc.store_scatter(cdloc, [ph * STG + off + iota],
                               jnp.full((16,), trash, i32), mask=tmask)

        @pl.when(pos > 0)
        def _():
            flush_start(nchunk, ph, ph * STG)

            @pl.when(nchunk >= 1)
            def _():
                flush_wait(1 - ph)

        nchunk = nchunk + jnp.where(pos > 0, 1, 0)

        total = jnp.maximum(_sc_ceil3(nchunk), NSLOT)
        ph2 = nchunk & 1

        @pl.when(nchunk >= 1)
        def _():
            flush_wait(1 - ph2)

        for j in range(CW // 16):
            csrc[pl.ds(j * 16, 16)] = jnp.zeros((16,), i32)
            cdloc[pl.ds(j * 16, 16)] = jnp.full((16,), trash, i32)

        def pad_body(i, carry):
            pltpu.async_copy(csrc.at[pl.ds(0, CW)], bsrc.at[c, s, i],
                             fsem.at[i & 1])
            pltpu.async_copy(cdloc.at[pl.ds(0, CW)], bdloc.at[c, s, i],
                             fsem.at[i & 1])
            flush_wait(i & 1)
            return carry

        lax.fori_loop(nchunk, total, pad_body, 0)

        cbuf[...] = jnp.broadcast_to(total, (16,)).astype(i32)
        pltpu.sync_copy(cbuf, counts.at[c, s])

        pltpu.sync_copy(hist, hshared.at[s])
        plsc.subcore_barrier()
        pltpu.sync_copy(hshared.at[:, pl.ds(s * tpr, tpr)], rbuf)
        half, quart = jnp.full((16,), 0.5), jnp.full((16,), 1.5)

        def _red(r, carry):
            acc16 = rbuf[0, r, :]
            for t in range(1, NS):
                acc16 = acc16 + rbuf[t, r, :]
            xi = plsc.bitcast(acc16, i32)
            yi = 0x5F3759DF - lax.shift_right_logical(xi, 1)
            y = plsc.bitcast(yi, jnp.float32)
            hx = acc16 * half
            for _ in range(3):
                y = y * (quart - hx * y * y)
            dbuf2[pl.ds(r * 16, 16)] = jnp.where(acc16 > 0, y, 0.0)
            return carry

        lax.fori_loop(0, tpr, _red, 0)
        goff = c * nh + s * tpt
        tail_n = nh - (NS - 1) * tpt

        @pl.when(s < NS - 1)
        def _():
            pltpu.sync_copy(dbuf2, dinv_out.at[pl.ds(goff, tpt)])

        @pl.when(s == NS - 1)
        def _():
            pltpu.sync_copy(dbuf2.at[pl.ds(0, tail_n)],
                            dinv_out.at[pl.ds(goff, tail_n)])

    return part_kernel


def _sc_ceil3(x):
    return (x + 2) - lax.rem(x + 2, 3)


def _make_agg_kernel(n, d, nh, acc_rows, tpt, capc):
    mesh = plsc.VectorSubcoreMesh(core_axis_name="c", subcore_axis_name="s", num_cores=NC, num_subcores=NS)
    i32 = jnp.int32

    @functools.partial(
        pl.kernel,
        out_type=jax.ShapeDtypeStruct((NC, acc_rows, d), jnp.float32),
        mesh=mesh,
        scratch_types=[
            pltpu.VMEM_SHARED((acc_rows, d), jnp.float32),
            pltpu.VMEM((NSLOT, CW), i32),
            pltpu.VMEM((NSLOT, CW), i32),
            pltpu.VMEM((NSLOT, CW, d), jnp.float32),
            pltpu.VMEM((ZR, d), jnp.float32),
            pltpu.VMEM((16,), i32),
            pltpu.SemaphoreType.DMA((NSLOT,)),
            pltpu.SemaphoreType.DMA((NSLOT,)),
            pltpu.SemaphoreType.DMA((NSLOT,)),
        ],
        compiler_params=pltpu.CompilerParams(use_tc_tiling_on_sc=False),
    )
    def agg_kernel(bsrc, bdloc, cnts_hbm, hs_hbm, out_hbm,
                   acc, sbuf, dbuf, rows_v, zbuf, cbuf,
                   idx_sem, gat_sem, sct_sem):
        c = lax.axis_index("c")
        s = lax.axis_index("s")

        pltpu.sync_copy(cnts_hbm.at[c, s], cbuf)
        nblk = cbuf[...][0]

        _zero_acc(acc, zbuf, s, tpt, d)
        plsc.subcore_barrier()

        def idx_start(b, slot):
            pltpu.async_copy(bsrc.at[c, s, b], sbuf.at[slot],
                             idx_sem.at[slot])
            pltpu.async_copy(bdloc.at[c, s, b], dbuf.at[slot],
                             idx_sem.at[slot])

        def idx_wait(slot):
            pltpu.make_async_copy(bsrc.at[c, s, 0], sbuf.at[slot],
                                  idx_sem.at[slot]).wait()
            pltpu.make_async_copy(bdloc.at[c, s, 0], dbuf.at[slot],
                                  idx_sem.at[slot]).wait()

        def gat_start(slot):
            pltpu.async_copy(hs_hbm.at[sbuf.at[slot]], rows_v.at[slot],
                             gat_sem.at[slot])

        def gat_wait(slot):
            pltpu.make_async_copy(hs_hbm.at[sbuf.at[slot]], rows_v.at[slot],
                                  gat_sem.at[slot]).wait()

        def sct_start(slot):
            pltpu.async_copy(rows_v.at[slot], acc.at[dbuf.at[slot]],
                             sct_sem.at[slot], add=True)

        def sct_wait(slot):
            pltpu.make_async_copy(rows_v.at[slot], acc.at[dbuf.at[slot]],
                                  sct_sem.at[slot]).wait()

        idx_start(0, 0)
        idx_start(1, 1)
        idx_wait(0)
        gat_start(0)

        def body(g, carry):
            for k in range(NSLOT):
                b = g * NSLOT + k
                cur = k
                nxt = (k + 1) % NSLOT
                pf = (k + 2) % NSLOT

                @pl.when(b + 2 < nblk)
                def _():
                    idx_start(b + 2, pf)

                @pl.when(b + 1 < nblk)
                def _():
                    idx_wait(nxt)

                    @pl.when(b >= 2)
                    def _():
                        sct_wait(nxt)

                    gat_start(nxt)

                gat_wait(cur)
                sct_start(cur)
            return carry

        lax.fori_loop(0, nblk // NSLOT, body, 0)
        for k in range(NSLOT):
            sct_wait(k)
        plsc.subcore_barrier()
        pltpu.sync_copy(acc.at[pl.ds(s * tpt, tpt)],
                        out_hbm.at[c, pl.ds(s * tpt, tpt)])

    return agg_kernel


def _tc_first(dinv, x, w, blk):
    n, d = x.shape

    def body(dinv_ref, x_ref, w_ref, hs_ref):
        hs_ref[...] = jnp.dot(x_ref[...], w_ref[...],
                              preferred_element_type=jnp.float32) * dinv_ref[...]

    return pl.pallas_call(
        body,
        grid=(n // blk,),
        in_specs=[
            pl.BlockSpec((blk, 1), lambda i: (i, 0)),
            pl.BlockSpec((blk, d), lambda i: (i, 0)),
            pl.BlockSpec((d, d), lambda i: (0, 0)),
        ],
        out_specs=pl.BlockSpec((blk, d), lambda i: (i, 0)),
        out_shape=jax.ShapeDtypeStruct((n, d), jnp.float32),
    )(dinv, x, w)


def _tc_mid(agg3, dinv, bias, res, w, scale, blk, nh):
    n, d = res.shape
    nb = nh // blk

    def body(agg_ref, dinv_ref, b_ref, res_ref, w_ref, out_ref, hs_ref):
        dv = dinv_ref[...]
        agg = agg_ref[...][0] * dv + b_ref[...]
        nrm = jnp.sqrt(jnp.sum(agg * agg, axis=-1, keepdims=True))
        h = agg / jnp.maximum(nrm, 1e-12)
        out_ref[...] = res_ref[...] + h * scale
        hs_ref[...] = jnp.dot(h, w_ref[...],
                              preferred_element_type=jnp.float32) * dv

    return pl.pallas_call(
        body,
        grid=(n // blk,),
        in_specs=[
            pl.BlockSpec((1, blk, d), lambda i: (i // nb, i % nb, 0)),
            pl.BlockSpec((blk, 1), lambda i: (i, 0)),
            pl.BlockSpec((1, d), lambda i: (0, 0)),
            pl.BlockSpec((blk, d), lambda i: (i, 0)),
            pl.BlockSpec((d, d), lambda i: (0, 0)),
        ],
        out_specs=[
            pl.BlockSpec((blk, d), lambda i: (i, 0)),
            pl.BlockSpec((blk, d), lambda i: (i, 0)),
        ],
        out_shape=[
            jax.ShapeDtypeStruct((n, d), jnp.float32),
            jax.ShapeDtypeStruct((n, d), jnp.float32),
        ],
    )(agg3, dinv, bias, res, w)


def _tc_last(agg3, dinv, bias, res, scale, blk, nh):
    n, d = res.shape
    nb = nh // blk

    def body(agg_ref, dinv_ref, b_ref, res_ref, out_ref):
        agg = agg_ref[...][0] * dinv_ref[...] + b_ref[...]
        nrm = jnp.sqrt(jnp.sum(agg * agg, axis=-1, keepdims=True))
        h = agg / jnp.maximum(nrm, 1e-12)
        out_ref[...] = res_ref[...] + h * scale

    return pl.pallas_call(
        body,
        grid=(n // blk,),
        in_specs=[
            pl.BlockSpec((1, blk, d), lambda i: (i // nb, i % nb, 0)),
            pl.BlockSpec((blk, 1), lambda i: (i, 0)),
            pl.BlockSpec((1, d), lambda i: (0, 0)),
            pl.BlockSpec((blk, d), lambda i: (i, 0)),
        ],
        out_specs=pl.BlockSpec((blk, d), lambda i: (i, 0)),
        out_shape=jax.ShapeDtypeStruct((n, d), jnp.float32),
    )(agg3, dinv, bias, res)


def kernel(x, edge_index, W, b):
    n, d = x.shape
    num_layers = W.shape[0]
    e = edge_index.shape[1]
    nh = n // NC
    tpt = _ceil_to(nh + 8, NS) // NS
    tpt = _ceil_to(tpt, ZR)
    acc_rows = NS * tpt

    src = edge_index[0].astype(jnp.int32)
    dst = edge_index[1].astype(jnp.int32)
    if e % CW != 0:
        pad = CW - e % CW
        src = jnp.concatenate([src, jnp.zeros((pad,), jnp.int32)])
        dst = jnp.concatenate([dst, jnp.full((pad,), n, jnp.int32)])
    rows = src.shape[0] // CW
    src2d = src.reshape(rows, CW)
    dst2d = dst.reshape(rows, CW)

    ptr = -(-rows // NS)
    capc = _ceil_to(ptr + 2, NSLOT)

    dinv1, bsrc, bdloc, counts = _make_partition_kernel(
        rows, n, nh, tpt, capc)(src2d, dst2d)
    dinv = dinv1.reshape(n, 1)

    blk = 1000
    agg_call = _make_agg_kernel(n, d, nh, acc_rows, tpt, capc)
    hs = _tc_first(dinv, x, W[0], blk)
    result = x
    for i in range(num_layers):
        agg3 = agg_call(bsrc, bdloc, counts, hs)
        scale = 1.0 / (i + 1)
        if i + 1 < num_layers:
            result, hs = _tc_mid(agg3, dinv, b[i].reshape(1, d), result,
                                 W[i + 1], scale, blk, nh)
        else:
            result = _tc_last(agg3, dinv, b[i].reshape(1, d), result,
                              scale, blk, nh)
    return result


# speedup vs baseline: 21.1384x; 1.0704x over previous
import functools

import jax
import jax.numpy as jnp
from jax import lax
from jax.experimental import pallas as pl
from jax.experimental.pallas import tpu as pltpu
from jax.experimental.pallas import tpu_sc as plsc

NC = 2
NS = 16
CW = 128
NSLOT = 3
ZR = 28
STG = 144


def _ceil_to(a, m):
    return -(-a // m) * m


def _zero_acc(acc, zbuf, s, tpt, width):
    def _fill(i, carry):
        for k in range(width // 16):
            zbuf[i, pl.ds(k * 16, 16)] = jnp.zeros((16,), jnp.float32)
        return carry

    lax.fori_loop(0, ZR, _fill, 0)
    for j in range(tpt // ZR):
        pltpu.sync_copy(zbuf, acc.at[pl.ds(s * tpt + j * ZR, ZR)])


def _make_partition_kernel(rows, n, nh, tpt, capc):
    ptr = -(-rows // NS)
    tpr = tpt // 16
    mesh = plsc.VectorSubcoreMesh(core_axis_name="c", subcore_axis_name="s", num_cores=NC, num_subcores=NS)
    i32 = jnp.int32

    @functools.partial(
        pl.kernel,
        out_type=[
            jax.ShapeDtypeStruct((n,), jnp.float32),
            jax.ShapeDtypeStruct((NC, NS, capc, CW), i32),
            jax.ShapeDtypeStruct((NC, NS, capc, CW), i32),
            jax.ShapeDtypeStruct((NC, NS, 16), i32),
        ],
        mesh=mesh,
        scratch_types=[
            pltpu.VMEM_SHARED((NS, tpt, 16), jnp.float32),
            pltpu.VMEM((tpt, 16), jnp.float32),
            pltpu.VMEM((NS, tpr, 16), jnp.float32),
            pltpu.VMEM((tpt,), jnp.float32),
            pltpu.VMEM((NSLOT, CW), i32),
            pltpu.VMEM((NSLOT, CW), i32),
            pltpu.VMEM((2 * STG,), i32),
            pltpu.VMEM((2 * STG,), i32),
            pltpu.VMEM((16,), i32),
            pltpu.SemaphoreType.DMA((NSLOT,)),
            pltpu.SemaphoreType.DMA((2,)),
        ],
        compiler_params=pltpu.CompilerParams(use_tc_tiling_on_sc=False,
                                             needs_layout_passes=False),
    )
    def part_kernel(src_hbm, dst_hbm, dinv_out, bsrc, bdloc, counts,
                    hshared, hist, rbuf, dbuf2, sbuf, dbuf, csrc, cdloc,
                    cbuf, idx_sem, fsem):
        c = lax.axis_index("c")
        s = lax.axis_index("s")
        base = c * nh
        trash = nh
        nb_t = (rows - s + NS - 1) // NS

        def _hz(i, carry):
            hist[i, :] = jnp.zeros((16,), jnp.float32)
            return carry

        lax.fori_loop(0, tpt, _hz, 0)

        def idx_start(b, slot):
            pltpu.async_copy(src_hbm.at[b * NS + s], sbuf.at[slot],
                             idx_sem.at[slot])
            pltpu.async_copy(dst_hbm.at[b * NS + s], dbuf.at[slot],
                             idx_sem.at[slot])

        def idx_wait(slot):
            pltpu.make_async_copy(src_hbm.at[s], sbuf.at[slot],
                                  idx_sem.at[slot]).wait()
            pltpu.make_async_copy(dst_hbm.at[s], dbuf.at[slot],
                                  idx_sem.at[slot]).wait()

        def flush_wait(ph):
            pltpu.make_async_copy(csrc.at[pl.ds(0, CW)], bsrc.at[c, s, 0],
                                  fsem.at[ph]).wait()
            pltpu.make_async_copy(csrc.at[pl.ds(0, CW)], bsrc.at[c, s, 0],
                                  fsem.at[ph]).wait()

        def flush_start(nchunk, ph, off):
            pltpu.async_copy(csrc.at[pl.ds(off, CW)],
                             bsrc.at[c, s, nchunk], fsem.at[ph])
            pltpu.async_copy(cdloc.at[pl.ds(off, CW)],
                             bdloc.at[c, s, nchunk], fsem.at[ph])

        idx_start(0, 0)
        idx_start(1, 1)

        def row_body(b, carry):
            pos, nchunk = carry

            @pl.when(b + 2 < nb_t)
            def _():
                idx_start(b + 2, (b + 2) % NSLOT)

            slot = b % NSLOT
            idx_wait(slot)
            for k in range(CW // 16):
                sv = sbuf[slot, pl.ds(k * 16, 16)]
                dv = dbuf[slot, pl.ds(k * 16, 16)]
                loc = dv - base
                ok = (loc >= 0) & (loc < nh)
                oki = jnp.where(ok, 1, 0)
                incl = plsc.cumsum(oki)
                cnt = jnp.sum(oki)
                ph = nchunk & 1
                widx = ph * STG + pos + incl - 1
                plsc.store_scatter(csrc, [widx], sv, mask=ok)
                plsc.store_scatter(cdloc, [widx], loc, mask=ok)
                cnts, lastm = plsc.scan_count(loc, mask=ok)
                plsc.addupdate_scatter(
                    hist, [lax.shift_right_logical(loc, 4), loc & 15],
                    cnts.astype(jnp.float32), mask=lastm)
                pos = pos + cnt
                flushed = pos >= CW

                @pl.when(flushed)
                def _():
                    flush_start(nchunk, ph, ph * STG)

                    @pl.when(nchunk >= 1)
                    def _():
                        flush_wait(1 - ph)
                    tail = csrc[pl.ds(ph * STG + CW, 16)]
                    csrc[pl.ds((1 - ph) * STG, 16)] = tail
                    tail2 = cdloc[pl.ds(ph * STG + CW, 16)]
                    cdloc[pl.ds((1 - ph) * STG, 16)] = tail2

                pos = jnp.where(flushed, pos - CW, pos)
                nchunk = nchunk + jnp.where(flushed, 1, 0)
            return pos, nchunk

        pos, nchunk = lax.fori_loop(
            0, nb_t, row_body, (jnp.int32(0), jnp.int32(0)))

        ph = nchunk & 1
        iota = lax.iota(i32, 16)
        for j in range(CW // 16):
            off = pos + j * 16
            tmask = (off + iota) < CW
            plsc.store_scatter(csrc, [ph * STG + off + iota],
                               jnp.zeros((16,), i32), mask=tmask)
            plsc.store_scatter(cdloc, [ph * STG + off + iota],
                               jnp.full((16,), trash, i32), mask=tmask)

        @pl.when(pos > 0)
        def _():
            flush_start(nchunk, ph, ph * STG)

            @pl.when(nchunk >= 1)
            def _():
                flush_wait(1 - ph)

        nchunk = nchunk + jnp.where(pos > 0, 1, 0)

        total = jnp.maximum(_sc_ceil3(nchunk), NSLOT)
        ph2 = nchunk & 1

        @pl.when(nchunk >= 1)
        def _():
            flush_wait(1 - ph2)

        for j in range(CW // 16):
            csrc[pl.ds(j * 16, 16)] = jnp.zeros((16,), i32)
            cdloc[pl.ds(j * 16, 16)] = jnp.full((16,), trash, i32)

        def pad_body(i, carry):
            pltpu.async_copy(csrc.at[pl.ds(0, CW)], bsrc.at[c, s, i],
                             fsem.at[i & 1])
            pltpu.async_copy(cdloc.at[pl.ds(0, CW)], bdloc.at[c, s, i],
                             fsem.at[i & 1])
            flush_wait(i & 1)
            return carry

        lax.fori_loop(nchunk, total, pad_body, 0)

        cbuf[...] = jnp.broadcast_to(total, (16,)).astype(i32)
        pltpu.sync_copy(cbuf, counts.at[c, s])

        pltpu.sync_copy(hist, hshared.at[s])
        plsc.subcore_barrier()
        pltpu.sync_copy(hshared.at[:, pl.ds(s * tpr, tpr)], rbuf)
        half, quart = jnp.full((16,), 0.5), jnp.full((16,), 1.5)

        def _red(r, carry):
            acc16 = rbuf[0, r, :]
            for t in range(1, NS):
                acc16 = acc16 + rbuf[t, r, :]
            xi = plsc.bitcast(acc16, i32)
            yi = 0x5F3759DF - lax.shift_right_logical(xi, 1)
            y = plsc.bitcast(yi, jnp.float32)
            hx = acc16 * half
            for _ in range(3):
                y = y * (quart - hx * y * y)
            dbuf2[pl.ds(r * 16, 16)] = jnp.where(acc16 > 0, y, 0.0)
            return carry

        lax.fori_loop(0, tpr, _red, 0)
        goff = c * nh + s * tpt
        tail_n = nh - (NS - 1) * tpt

        @pl.when(s < NS - 1)
        def _():
            pltpu.sync_copy(dbuf2, dinv_out.at[pl.ds(goff, tpt)])

        @pl.when(s == NS - 1)
        def _():
            pltpu.sync_copy(dbuf2.at[pl.ds(0, tail_n)],
                            dinv_out.at[pl.ds(goff, tail_n)])

    return part_kernel


def _sc_ceil3(x):
    return (x + 2) - lax.rem(x + 2, 3)


def _make_agg_kernel(n, d, nh, acc_rows, tpt, capc):
    mesh = plsc.VectorSubcoreMesh(core_axis_name="c", subcore_axis_name="s", num_cores=NC, num_subcores=NS)
    i32 = jnp.int32

    @functools.partial(
        pl.kernel,
        out_type=jax.ShapeDtypeStruct((NC, acc_rows, d), jnp.float32),
        mesh=mesh,
        scratch_types=[
            pltpu.VMEM_SHARED((acc_rows, d), jnp.float32),
            pltpu.VMEM((NSLOT, CW), i32),
            pltpu.VMEM((NSLOT, CW), i32),
            pltpu.VMEM((NSLOT, CW, d), jnp.float32),
            pltpu.VMEM((ZR, d), jnp.float32),
            pltpu.VMEM((16,), i32),
            pltpu.SemaphoreType.DMA((NSLOT,)),
            pltpu.SemaphoreType.DMA((NSLOT,)),
            pltpu.SemaphoreType.DMA((NSLOT,)),
        ],
        compiler_params=pltpu.CompilerParams(use_tc_tiling_on_sc=False),
    )
    def agg_kernel(bsrc, bdloc, cnts_hbm, hs_hbm, out_hbm,
                   acc, sbuf, dbuf, rows_v, zbuf, cbuf,
                   idx_sem, gat_sem, sct_sem):
        c = lax.axis_index("c")
        s = lax.axis_index("s")

        pltpu.sync_copy(cnts_hbm.at[c, s], cbuf)
        nblk = cbuf[...][0]

        _zero_acc(acc, zbuf, s, tpt, d)
        plsc.subcore_barrier()

        def idx_start(b, slot):
            pltpu.async_copy(bsrc.at[c, s, b], sbuf.at[slot],
                             idx_sem.at[slot])
            pltpu.async_copy(bdloc.at[c, s, b], dbuf.at[slot],
                             idx_sem.at[slot])

        def idx_wait(slot):
            pltpu.make_async_copy(bsrc.at[c, s, 0], sbuf.at[slot],
                                  idx_sem.at[slot]).wait()
            pltpu.make_async_copy(bdloc.at[c, s, 0], dbuf.at[slot],
                                  idx_sem.at[slot]).wait()

        def gat_start(slot):
            pltpu.async_copy(hs_hbm.at[sbuf.at[slot]], rows_v.at[slot],
                             gat_sem.at[slot])

        def gat_wait(slot):
            pltpu.make_async_copy(hs_hbm.at[sbuf.at[slot]], rows_v.at[slot],
                                  gat_sem.at[slot]).wait()

        def sct_start(slot):
            pltpu.async_copy(rows_v.at[slot], acc.at[dbuf.at[slot]],
                             sct_sem.at[slot], add=True)

        def sct_wait(slot):
            pltpu.make_async_copy(rows_v.at[slot], acc.at[dbuf.at[slot]],
                                  sct_sem.at[slot]).wait()

        idx_start(0, 0)
        idx_start(1, 1)
        idx_wait(0)
        gat_start(0)

        def body(g, carry):
            for k in range(NSLOT):
                b = g * NSLOT + k
                cur = k
                nxt = (k + 1) % NSLOT
                pf = (k + 2) % NSLOT

                @pl.when(b + 2 < nblk)
                def _():
                    idx_start(b + 2, pf)

                @pl.when(b + 1 < nblk)
                def _():
                    idx_wait(nxt)

                    @pl.when(b >= 2)
                    def _():
                        sct_wait(nxt)

                    gat_start(nxt)

                gat_wait(cur)
                sct_start(cur)
            return carry

        lax.fori_loop(0, nblk // NSLOT, body, 0)
        for k in range(NSLOT):
            sct_wait(k)
        plsc.subcore_barrier()
        pltpu.sync_copy(acc.at[pl.ds(s * tpt, tpt)],
                        out_hbm.at[c, pl.ds(s * tpt, tpt)])

    return agg_kernel


def _tc_first(dinv, x, w, blk):
    n, d = x.shape

    def body(dinv_ref, x_ref, w_ref, hs_ref):
        hs_ref[...] = jnp.dot(x_ref[...], w_ref[...],
                              preferred_element_type=jnp.float32) * dinv_ref[...]

    return pl.pallas_call(
        body,
        grid=(n // blk,),
        in_specs=[
            pl.BlockSpec((blk, 1), lambda i: (i, 0)),
            pl.BlockSpec((blk, d), lambda i: (i, 0)),
            pl.BlockSpec((d, d), lambda i: (0, 0)),
        ],
        out_specs=pl.BlockSpec((blk, d), lambda i: (i, 0)),
        out_shape=jax.ShapeDtypeStruct((n, d), jnp.float32),
    )(dinv, x, w)


def _tc_mid(agg3, dinv, bias, res, w, scale, blk, nh):
    n, d = res.shape
    nb = nh // blk

    def body(agg_ref, dinv_ref, b_ref, res_ref, w_ref, out_ref, hs_ref):
        dv = dinv_ref[...]
        agg = agg_ref[...][0] * dv + b_ref[...]
        nrm = jnp.sqrt(jnp.sum(agg * agg, axis=-1, keepdims=True))
        h = agg / jnp.maximum(nrm, 1e-12)
        out_ref[...] = res_ref[...] + h * scale
        hs_ref[...] = jnp.dot(h, w_ref[...],
                              preferred_element_type=jnp.float32) * dv

    return pl.pallas_call(
        body,
        grid=(n // blk,),
        in_specs=[
            pl.BlockSpec((1, blk, d), lambda i: (i // nb, i % nb, 0)),
            pl.BlockSpec((blk, 1), lambda i: (i, 0)),
            pl.BlockSpec((1, d), lambda i: (0, 0)),
            pl.BlockSpec((blk, d), lambda i: (i, 0)),
            pl.BlockSpec((d, d), lambda i: (0, 0)),
        ],
        out_specs=[
            pl.BlockSpec((blk, d), lambda i: (i, 0)),
            pl.BlockSpec((blk, d), lambda i: (i, 0)),
        ],
        out_shape=[
            jax.ShapeDtypeStruct((n, d), jnp.float32),
            jax.ShapeDtypeStruct((n, d), jnp.float32),
        ],
    )(agg3, dinv, bias, res, w)


def _tc_last(agg3, dinv, bias, res, scale, blk, nh):
    n, d = res.shape
    nb = nh // blk

    def body(agg_ref, dinv_ref, b_ref, res_ref, out_ref):
        agg = agg_ref[...][0] * dinv_ref[...] + b_ref[...]
        nrm = jnp.sqrt(jnp.sum(agg * agg, axis=-1, keepdims=True))
        h = agg / jnp.maximum(nrm, 1e-12)
        out_ref[...] = res_ref[...] + h * scale

    return pl.pallas_call(
        body,
        grid=(n // blk,),
        in_specs=[
            pl.BlockSpec((1, blk, d), lambda i: (i // nb, i % nb, 0)),
            pl.BlockSpec((blk, 1), lambda i: (i, 0)),
            pl.BlockSpec((1, d), lambda i: (0, 0)),
            pl.BlockSpec((blk, d), lambda i: (i, 0)),
        ],
        out_specs=pl.BlockSpec((blk, d), lambda i: (i, 0)),
        out_shape=jax.ShapeDtypeStruct((n, d), jnp.float32),
    )(agg3, dinv, bias, res)


def kernel(x, edge_index, W, b):
    n, d = x.shape
    num_layers = W.shape[0]
    e = edge_index.shape[1]
    nh = n // NC
    tpt = _ceil_to(nh + 8, NS) // NS
    tpt = _ceil_to(tpt, ZR)
    acc_rows = NS * tpt

    src = edge_index[0].astype(jnp.int32)
    dst = edge_index[1].astype(jnp.int32)
    if e % CW != 0:
        pad = CW - e % CW
        src = jnp.concatenate([src, jnp.zeros((pad,), jnp.int32)])
        dst = jnp.concatenate([dst, jnp.full((pad,), n, jnp.int32)])
    rows = src.shape[0] // CW
    src2d = src.reshape(rows, CW)
    dst2d = dst.reshape(rows, CW)

    ptr = -(-rows // NS)
    capc = _ceil_to(ptr + 2, 8 * NSLOT)

    dinv1, bsrc, bdloc, counts = _make_partition_kernel(
        rows, n, nh, tpt, capc)(src2d, dst2d)
    dinv = dinv1.reshape(n, 1)

    blk = 5000
    agg_call = _make_agg_kernel(n, d, nh, acc_rows, tpt, capc)
    hs = _tc_first(dinv, x, W[0], blk)
    result = x
    for i in range(num_layers):
        agg3 = agg_call(bsrc, bdloc, counts, hs)
        scale = 1.0 / (i + 1)
        if i + 1 < num_layers:
            result, hs = _tc_mid(agg3, dinv, b[i].reshape(1, d), result,
                                 W[i + 1], scale, blk, nh)
        else:
            result = _tc_last(agg3, dinv, b[i].reshape(1, d), result,
                              scale, blk, nh)
    return result
